# baseline (speedup 1.0000x reference)
import functools

import jax
import jax.numpy as jnp
from jax import lax
from jax.experimental import pallas as pl
from jax.experimental.pallas import tpu as pltpu
from jax.experimental.pallas import tpu_sc as plsc

HIDDEN = 64
OUT = 128
NW = 32
C = 128


def _tile_proj_body(tt, w, o):
    o[...] = jnp.dot(tt[...], w[...], preferred_element_type=jnp.float32)


def _tile_proj(tile_table, w_t):
    n = tile_table.shape[0]
    blk = 10000
    return pl.pallas_call(
        _tile_proj_body,
        grid=(n // blk,),
        in_specs=[pl.BlockSpec((blk, HIDDEN), lambda i: (i, 0)),
                  pl.BlockSpec((HIDDEN, OUT), lambda i: (0, 0))],
        out_specs=pl.BlockSpec((blk, OUT), lambda i: (i, 0)),
        out_shape=jax.ShapeDtypeStruct((n, OUT), jnp.float32),
    )(tile_table, w_t)


def _colrow_body(col, row, wc, wr, b, o):
    ce = jnp.dot(col[...], wc[...], preferred_element_type=jnp.float32)
    re = jnp.dot(row[...], wr[...], preferred_element_type=jnp.float32) + b[...]
    o[...] = ce[:, None, :] + re[None, :, :]


def _colrow_proj(col_table, row_table, wc, wr, b):
    wd, hd = col_table.shape[0], row_table.shape[0]
    blk = 40
    out = pl.pallas_call(
        _colrow_body,
        grid=(wd // blk,),
        in_specs=[pl.BlockSpec((blk, HIDDEN), lambda i: (i, 0)),
                  pl.BlockSpec((hd, HIDDEN), lambda i: (0, 0)),
                  pl.BlockSpec((HIDDEN, OUT), lambda i: (0, 0)),
                  pl.BlockSpec((HIDDEN, OUT), lambda i: (0, 0)),
                  pl.BlockSpec((1, OUT), lambda i: (0, 0))],
        out_specs=pl.BlockSpec((blk, hd, OUT), lambda i: (i, 0, 0)),
        out_shape=jax.ShapeDtypeStruct((wd, hd, OUT), jnp.float32),
    )(col_table, row_table, wc, wr, b.reshape(1, OUT))
    return out.reshape(wd * hd, OUT)


def _idx_body(hd, t, x, y, o):
    o[:, 0, :] = t[...]
    o[:, 1, :] = x[...] * hd + y[...]


def _idx_pack(tile_f, x_f, y_f, hd, nchunks):
    blk = 256
    return pl.pallas_call(
        functools.partial(_idx_body, hd),
        grid=(nchunks // blk,),
        in_specs=[pl.BlockSpec((blk, C), lambda i: (i, 0))] * 3,
        out_specs=pl.BlockSpec((blk, 2, C), lambda i: (i, 0, 0)),
        out_shape=jax.ShapeDtypeStruct((nchunks, 2, C), jnp.int32),
    )(tile_f.reshape(nchunks, C), x_f.reshape(nchunks, C),
      y_f.reshape(nchunks, C))


NBUF = 6


def _sc_body(chunks_pw, idx_hbm, tp_hbm, cr_hbm, out_hbm,
             ibuf, *rest):
    wid = lax.axis_index("s") * 2 + lax.axis_index("c")
    c0 = wid * chunks_pw
    bufs = rest[:NBUF]
    sems = rest[NBUF:]
    isems, tsems = sems[0:NBUF], sems[NBUF:2 * NBUF]
    csems, wsems = sems[2 * NBUF:3 * NBUF], sems[3 * NBUF:4 * NBUF]

    def w_copy(g, j):
        return pltpu.make_async_copy(
            bufs[j], out_hbm.at[pl.ds((c0 + g) * C, C)], wsems[j])

    for s in range(NBUF):
        pltpu.async_copy(idx_hbm.at[c0 + s], ibuf.at[s], isems[s])

    def step(g, j):
        jm1, jm2 = (j - 1) % NBUF, (j - 2) % NBUF

        @pl.when((g >= NBUF) & (g < chunks_pw + NBUF))
        def _():
            w_copy(g - NBUF, j).wait()

        @pl.when(g < chunks_pw)
        def _():
            pltpu.make_async_copy(
                idx_hbm.at[c0 + g], ibuf.at[j], isems[j]).wait()
            pltpu.async_copy(tp_hbm.at[ibuf.at[j, 0]], bufs[j], tsems[j])

        @pl.when((g >= 1) & (g < chunks_pw + 1))
        def _():
            pltpu.make_async_copy(
                tp_hbm.at[ibuf.at[jm1, 0]], bufs[jm1], tsems[jm1]).wait()
            pltpu.async_copy(
                cr_hbm.at[ibuf.at[jm1, 1]], bufs[jm1], csems[jm1], add=True)

        @pl.when((g >= 2) & (g < chunks_pw + 2))
        def _():
            pltpu.make_async_copy(
                cr_hbm.at[ibuf.at[jm2, 1]], bufs[jm2], csems[jm2]).wait()
            pltpu.async_copy(
                bufs[jm2], out_hbm.at[pl.ds((c0 + g - 2) * C, C)], wsems[jm2])

            @pl.when(g + NBUF - 2 < chunks_pw)
            def _():
                pltpu.async_copy(
                    idx_hbm.at[c0 + g + NBUF - 2], ibuf.at[jm2], isems[jm2])

    def body(p, carry):
        for j in range(NBUF):
            step(NBUF * p + j, j)
        return carry
    lax.fori_loop(0, (chunks_pw + 2 * NBUF - 1) // NBUF, body, 0)


def _sc_call(idx3, tp, cr):
    nchunks = idx3.shape[0]
    tokens = nchunks * C
    mesh = plsc.VectorSubcoreMesh(core_axis_name="c", subcore_axis_name="s")
    kfn = pl.kernel(
        functools.partial(_sc_body, nchunks // NW),
        out_type=jax.ShapeDtypeStruct((tokens, OUT), jnp.float32),
        mesh=mesh,
        scratch_types=[pltpu.VMEM((NBUF, 2, C), jnp.int32)]
        + [pltpu.VMEM((C, OUT), jnp.float32)] * NBUF
        + [pltpu.SemaphoreType.DMA] * (4 * NBUF),
    )
    return kfn(idx3, tp, cr)


def kernel(tile, x, y, tile_table, col_table, row_table, W, b):
    bsz, seq = tile.shape
    hd = row_table.shape[0]
    tp = _tile_proj(tile_table, W[:HIDDEN])
    cr = _colrow_proj(col_table, row_table, W[HIDDEN:2 * HIDDEN],
                      W[2 * HIDDEN:], b)
    nchunks = (bsz * seq) // C
    idx3 = _idx_pack(tile.reshape(-1), x.reshape(-1), y.reshape(-1),
                     hd, nchunks)
    out = _sc_call(idx3, tp, cr)
    return out.reshape(bsz, seq, OUT)


# speedup vs baseline: 1.1176x; 1.1176x over previous
import functools

import jax
import jax.numpy as jnp
from jax import lax
from jax.experimental import pallas as pl
from jax.experimental.pallas import tpu as pltpu
from jax.experimental.pallas import tpu_sc as plsc

HIDDEN = 64
OUT = 128
NW = 32
C = 128


def _tile_proj_body(tt, w, o):
    o[...] = jnp.dot(tt[...], w[...], preferred_element_type=jnp.float32)


def _tile_proj(tile_table, w_t):
    n = tile_table.shape[0]
    blk = 10000
    return pl.pallas_call(
        _tile_proj_body,
        grid=(n // blk,),
        in_specs=[pl.BlockSpec((blk, HIDDEN), lambda i: (i, 0)),
                  pl.BlockSpec((HIDDEN, OUT), lambda i: (0, 0))],
        out_specs=pl.BlockSpec((blk, OUT), lambda i: (i, 0)),
        out_shape=jax.ShapeDtypeStruct((n, OUT), jnp.float32),
    )(tile_table, w_t)


def _colrow_body(wd, col, row, wc, wr, b, o):
    hd = row.shape[0]
    o[0:wd] = jnp.dot(col[...], wc[...], preferred_element_type=jnp.float32)
    o[wd:wd + hd] = (
        jnp.dot(row[...], wr[...], preferred_element_type=jnp.float32)
        + b[...])


def _colrow_proj(col_table, row_table, wc, wr, b):
    wd, hd = col_table.shape[0], row_table.shape[0]
    return pl.pallas_call(
        functools.partial(_colrow_body, wd),
        out_shape=jax.ShapeDtypeStruct((wd + hd, OUT), jnp.float32),
    )(col_table, row_table, wc, wr, b.reshape(1, OUT))


def _idx_body(wd, t, x, y, o):
    o[:, 0, :] = t[...]
    o[:, 1, :] = x[...]
    o[:, 2, :] = y[...] + wd


def _idx_pack(tile_f, x_f, y_f, wd, nchunks):
    blk = 256
    return pl.pallas_call(
        functools.partial(_idx_body, wd),
        grid=(nchunks // blk,),
        in_specs=[pl.BlockSpec((blk, C), lambda i: (i, 0))] * 3,
        out_specs=pl.BlockSpec((blk, 3, C), lambda i: (i, 0, 0)),
        out_shape=jax.ShapeDtypeStruct((nchunks, 3, C), jnp.int32),
    )(tile_f.reshape(nchunks, C), x_f.reshape(nchunks, C),
      y_f.reshape(nchunks, C))


NBUF = 4


def _sc_body(chunks_pw, idx_hbm, tp_hbm, cr_hbm, out_hbm,
             ibuf, spt, *rest):
    wid = lax.axis_index("s") * 2 + lax.axis_index("c")
    c0 = wid * chunks_pw
    bufs = rest[:NBUF]
    sems = rest[NBUF:]
    isems, tsems = sems[0:NBUF], sems[NBUF:2 * NBUF]
    xsems, ysems = sems[2 * NBUF:3 * NBUF], sems[3 * NBUF:4 * NBUF]
    wsems = sems[4 * NBUF:5 * NBUF]

    @pl.when(lax.axis_index("s") == 0)
    def _():
        pltpu.sync_copy(cr_hbm, spt)
    plsc.subcore_barrier()

    def w_copy(g, j):
        return pltpu.make_async_copy(
            bufs[j], out_hbm.at[pl.ds((c0 + g) * C, C)], wsems[j])

    for s in range(NBUF):
        pltpu.async_copy(idx_hbm.at[c0 + s], ibuf.at[s], isems[s])

    def step(g, j):
        jm1, jm2 = (j - 1) % NBUF, (j - 2) % NBUF

        @pl.when((g >= NBUF) & (g < chunks_pw + NBUF))
        def _():
            w_copy(g - NBUF, j).wait()

        @pl.when(g < chunks_pw)
        def _():
            pltpu.make_async_copy(
                idx_hbm.at[c0 + g], ibuf.at[j], isems[j]).wait()
            pltpu.async_copy(tp_hbm.at[ibuf.at[j, 0]], bufs[j], tsems[j])

        @pl.when((g >= 1) & (g < chunks_pw + 1))
        def _():
            pltpu.make_async_copy(
                tp_hbm.at[ibuf.at[jm1, 0]], bufs[jm1], tsems[jm1]).wait()
            pltpu.async_copy(
                spt.at[ibuf.at[jm1, 1]], bufs[jm1], xsems[jm1], add=True)
            pltpu.async_copy(
                spt.at[ibuf.at[jm1, 2]], bufs[jm1], ysems[jm1], add=True)

        @pl.when((g >= 2) & (g < chunks_pw + 2))
        def _():
            pltpu.make_async_copy(
                spt.at[ibuf.at[jm2, 1]], bufs[jm2], xsems[jm2]).wait()
            pltpu.make_async_copy(
                spt.at[ibuf.at[jm2, 2]], bufs[jm2], ysems[jm2]).wait()
            pltpu.async_copy(
                bufs[jm2], out_hbm.at[pl.ds((c0 + g - 2) * C, C)], wsems[jm2])

            @pl.when(g + NBUF - 2 < chunks_pw)
            def _():
                pltpu.async_copy(
                    idx_hbm.at[c0 + g + NBUF - 2], ibuf.at[jm2], isems[jm2])

    def body(p, carry):
        for j in range(NBUF):
            step(NBUF * p + j, j)
        return carry
    lax.fori_loop(0, (chunks_pw + 2 * NBUF - 1) // NBUF, body, 0)


def _sc_call(idx3, tp, cr):
    nchunks = idx3.shape[0]
    tokens = nchunks * C
    mesh = plsc.VectorSubcoreMesh(core_axis_name="c", subcore_axis_name="s")
    kfn = pl.kernel(
        functools.partial(_sc_body, nchunks // NW),
        out_type=jax.ShapeDtypeStruct((tokens, OUT), jnp.float32),
        mesh=mesh,
        scratch_types=[
            pltpu.VMEM((NBUF, 3, C), jnp.int32),
            pltpu.VMEM_SHARED(cr.shape, jnp.float32),
        ]
        + [pltpu.VMEM((C, OUT), jnp.float32)] * NBUF
        + [pltpu.SemaphoreType.DMA] * (5 * NBUF),
    )
    return kfn(idx3, tp, cr)


def kernel(tile, x, y, tile_table, col_table, row_table, W, b):
    bsz, seq = tile.shape
    wd = col_table.shape[0]
    tp = _tile_proj(tile_table, W[:HIDDEN])
    cr = _colrow_proj(col_table, row_table, W[HIDDEN:2 * HIDDEN],
                      W[2 * HIDDEN:], b)
    nchunks = (bsz * seq) // C
    idx3 = _idx_pack(tile.reshape(-1), x.reshape(-1), y.reshape(-1),
                     wd, nchunks)
    out = _sc_call(idx3, tp, cr)
    return out.reshape(bsz, seq, OUT)


# speedup vs baseline: 1.1181x; 1.0005x over previous
import functools

import jax
import jax.numpy as jnp
from jax import lax
from jax.experimental import pallas as pl
from jax.experimental.pallas import tpu as pltpu
from jax.experimental.pallas import tpu_sc as plsc

HIDDEN = 64
OUT = 128
NW = 32
C = 128


def _tile_proj_body(tt, w, o):
    o[...] = jnp.dot(tt[...], w[...], preferred_element_type=jnp.float32)


def _tile_proj(tile_table, w_t):
    n = tile_table.shape[0]
    blk = 10000
    return pl.pallas_call(
        _tile_proj_body,
        grid=(n // blk,),
        in_specs=[pl.BlockSpec((blk, HIDDEN), lambda i: (i, 0)),
                  pl.BlockSpec((HIDDEN, OUT), lambda i: (0, 0))],
        out_specs=pl.BlockSpec((blk, OUT), lambda i: (i, 0)),
        out_shape=jax.ShapeDtypeStruct((n, OUT), jnp.float32),
    )(tile_table, w_t)


def _colrow_body(wd, col, row, wc, wr, b, o):
    hd = row.shape[0]
    o[0:wd] = jnp.dot(col[...], wc[...], preferred_element_type=jnp.float32)
    o[wd:wd + hd] = (
        jnp.dot(row[...], wr[...], preferred_element_type=jnp.float32)
        + b[...])


def _colrow_proj(col_table, row_table, wc, wr, b):
    wd, hd = col_table.shape[0], row_table.shape[0]
    return pl.pallas_call(
        functools.partial(_colrow_body, wd),
        out_shape=jax.ShapeDtypeStruct((wd + hd, OUT), jnp.float32),
    )(col_table, row_table, wc, wr, b.reshape(1, OUT))


def _idx_body(wd, t, x, y, o):
    o[:, 0, :] = t[...]
    o[:, 1, :] = x[...]
    o[:, 2, :] = y[...] + wd


def _idx_pack(tile_f, x_f, y_f, wd, nchunks):
    blk = 256
    return pl.pallas_call(
        functools.partial(_idx_body, wd),
        grid=(nchunks // blk,),
        in_specs=[pl.BlockSpec((blk, C), lambda i: (i, 0))] * 3,
        out_specs=pl.BlockSpec((blk, 3, C), lambda i: (i, 0, 0)),
        out_shape=jax.ShapeDtypeStruct((nchunks, 3, C), jnp.int32),
    )(tile_f.reshape(nchunks, C), x_f.reshape(nchunks, C),
      y_f.reshape(nchunks, C))


NBUF = 6


def _sc_body(chunks_pw, idx_hbm, tp_hbm, cr_hbm, out_hbm,
             ibuf, spt, *rest):
    wid = lax.axis_index("s") * 2 + lax.axis_index("c")
    c0 = wid * chunks_pw
    bufs = rest[:NBUF]
    sems = rest[NBUF:]
    isems, tsems = sems[0:NBUF], sems[NBUF:2 * NBUF]
    xsems, ysems = sems[2 * NBUF:3 * NBUF], sems[3 * NBUF:4 * NBUF]
    wsems = sems[4 * NBUF:5 * NBUF]

    @pl.when(lax.axis_index("s") == 0)
    def _():
        pltpu.sync_copy(cr_hbm, spt)
    plsc.subcore_barrier()

    def w_copy(g, j):
        return pltpu.make_async_copy(
            bufs[j], out_hbm.at[pl.ds((c0 + g) * C, C)], wsems[j])

    for s in range(NBUF):
        pltpu.async_copy(idx_hbm.at[c0 + s], ibuf.at[s], isems[s])

    def step(g, j):
        jm2, jm4 = (j - 2) % NBUF, (j - 4) % NBUF

        @pl.when((g >= NBUF) & (g < chunks_pw + NBUF))
        def _():
            w_copy(g - NBUF, j).wait()

        @pl.when(g < chunks_pw)
        def _():
            pltpu.make_async_copy(
                idx_hbm.at[c0 + g], ibuf.at[j], isems[j]).wait()
            pltpu.async_copy(tp_hbm.at[ibuf.at[j, 0]], bufs[j], tsems[j])

        @pl.when((g >= 2) & (g < chunks_pw + 2))
        def _():
            pltpu.make_async_copy(
                tp_hbm.at[ibuf.at[jm2, 0]], bufs[jm2], tsems[jm2]).wait()
            pltpu.async_copy(
                spt.at[ibuf.at[jm2, 1]], bufs[jm2], xsems[jm2], add=True)
            pltpu.async_copy(
                spt.at[ibuf.at[jm2, 2]], bufs[jm2], ysems[jm2], add=True)

        @pl.when((g >= 4) & (g < chunks_pw + 4))
        def _():
            pltpu.make_async_copy(
                spt.at[ibuf.at[jm4, 1]], bufs[jm4], xsems[jm4]).wait()
            pltpu.make_async_copy(
                spt.at[ibuf.at[jm4, 2]], bufs[jm4], ysems[jm4]).wait()
            pltpu.async_copy(
                bufs[jm4], out_hbm.at[pl.ds((c0 + g - 4) * C, C)], wsems[jm4])

            @pl.when(g + NBUF - 4 < chunks_pw)
            def _():
                pltpu.async_copy(
                    idx_hbm.at[c0 + g + NBUF - 4], ibuf.at[jm4], isems[jm4])

    def body(p, carry):
        for j in range(NBUF):
            step(NBUF * p + j, j)
        return carry
    lax.fori_loop(0, (chunks_pw + 2 * NBUF - 1) // NBUF, body, 0)


def _sc_call(idx3, tp, cr):
    nchunks = idx3.shape[0]
    tokens = nchunks * C
    mesh = plsc.VectorSubcoreMesh(core_axis_name="c", subcore_axis_name="s")
    kfn = pl.kernel(
        functools.partial(_sc_body, nchunks // NW),
        out_type=jax.ShapeDtypeStruct((tokens, OUT), jnp.float32),
        mesh=mesh,
        scratch_types=[
            pltpu.VMEM((NBUF, 3, C), jnp.int32),
            pltpu.VMEM_SHARED(cr.shape, jnp.float32),
        ]
        + [pltpu.VMEM((C, OUT), jnp.float32)] * NBUF
        + [pltpu.SemaphoreType.DMA] * (5 * NBUF),
    )
    return kfn(idx3, tp, cr)


def kernel(tile, x, y, tile_table, col_table, row_table, W, b):
    bsz, seq = tile.shape
    wd = col_table.shape[0]
    tp = _tile_proj(tile_table, W[:HIDDEN])
    cr = _colrow_proj(col_table, row_table, W[HIDDEN:2 * HIDDEN],
                      W[2 * HIDDEN:], b)
    nchunks = (bsz * seq) // C
    idx3 = _idx_pack(tile.reshape(-1), x.reshape(-1), y.reshape(-1),
                     wd, nchunks)
    out = _sc_call(idx3, tp, cr)
    return out.reshape(bsz, seq, OUT)
